# baseline (device time: 4252009 ns/iter reference)
import jax
import jax.numpy as jnp
from jax import lax
from jax.experimental import pallas as pl
from jax.experimental.pallas import tpu as pltpu

N_CHUNKS = 16


def kernel(x):
    m, n = x.shape
    half = m // 2
    ch = half // N_CHUNKS

    def body(x_ref, out_ref, local_sem, x_send, x_recv, y_send, y_recv):
        my_x = lax.axis_index("x")
        my_y = lax.axis_index("y")
        other_x = 1 - my_x

        local = pltpu.make_async_copy(
            x_ref, out_ref.at[pl.ds(my_x * m, m), :], local_sem
        )
        local.start()

        x_rdmas = []
        for c in range(N_CHUNKS):
            row = my_y * half + c * ch
            r = pltpu.make_async_remote_copy(
                src_ref=x_ref.at[pl.ds(row, ch), :],
                dst_ref=out_ref.at[pl.ds(my_x * m + row, ch), :],
                send_sem=x_send.at[c],
                recv_sem=x_recv.at[c],
                device_id=(other_x, my_y),
                device_id_type=pl.DeviceIdType.MESH,
            )
            r.start()
            x_rdmas.append(r)

        y_rdmas = []
        for c in range(N_CHUNKS):
            x_rdmas[c].wait_recv()
            row = other_x * m + my_y * half + c * ch
            f = pltpu.make_async_remote_copy(
                src_ref=out_ref.at[pl.ds(row, ch), :],
                dst_ref=out_ref.at[pl.ds(row, ch), :],
                send_sem=y_send.at[c],
                recv_sem=y_recv.at[c],
                device_id=(my_x, 1 - my_y),
                device_id_type=pl.DeviceIdType.MESH,
            )
            f.start()
            y_rdmas.append(f)

        for c in range(N_CHUNKS):
            y_rdmas[c].wait_recv()
        for c in range(N_CHUNKS):
            x_rdmas[c].wait_send()
            y_rdmas[c].wait_send()
        local.wait()

    return pl.pallas_call(
        body,
        out_shape=jax.ShapeDtypeStruct((2 * m, n), x.dtype),
        in_specs=[pl.BlockSpec(memory_space=pl.ANY)],
        out_specs=pl.BlockSpec(memory_space=pl.ANY),
        scratch_shapes=[
            pltpu.SemaphoreType.DMA,
            pltpu.SemaphoreType.DMA((N_CHUNKS,)),
            pltpu.SemaphoreType.DMA((N_CHUNKS,)),
            pltpu.SemaphoreType.DMA((N_CHUNKS,)),
            pltpu.SemaphoreType.DMA((N_CHUNKS,)),
        ],
    )(x)
